# baseline (speedup 1.0000x reference)
import functools

import jax
import jax.numpy as jnp
from jax import lax
from jax.experimental import pallas as pl
from jax.experimental.pallas import tpu as pltpu
from jax.experimental.pallas import tpu_sc as plsc

N_TILES = 32
LANES = 16


def _row_block(n):
    for rb in (2000, 1000, 500, 250, 200, 125, 100, 50, 25, 10, 8, 5, 4, 2, 1):
        if n % rb == 0 and rb % 8 == 0 or n % rb == 0 and rb < 8:
            return rb
    return 1


def _pre_body(x_ref, w1t_ref, w2t_ref, t_ref, u_ref):
    xb = x_ref[...]
    d = xb.shape[1]
    t_ref[:, :d] = xb
    t_ref[:, d:] = jnp.dot(xb, w2t_ref[...], preferred_element_type=jnp.float32)
    u_ref[...] = jnp.dot(xb, w1t_ref[...], preferred_element_type=jnp.float32)


def _tc_pre(x, w1t, w2t):
    n, d = x.shape
    rb = _row_block(n)
    return pl.pallas_call(
        _pre_body,
        grid=(n // rb,),
        in_specs=[
            pl.BlockSpec((rb, d), lambda i: (i, 0)),
            pl.BlockSpec((d, d), lambda i: (0, 0)),
            pl.BlockSpec((d, d), lambda i: (0, 0)),
        ],
        out_specs=[
            pl.BlockSpec((rb, 2 * d), lambda i: (i, 0)),
            pl.BlockSpec((rb, d), lambda i: (i, 0)),
        ],
        out_shape=[
            jax.ShapeDtypeStruct((n, 2 * d), jnp.float32),
            jax.ShapeDtypeStruct((n, d), jnp.float32),
        ],
    )(x, w1t, w2t)


def _make_sc(e_pad, n_pad, d, npt, chunk, grp):
    nb = d // LANES
    n_chunks = e_pad // chunk
    cpad = npt + LANES
    mesh = plsc.VectorSubcoreMesh(core_axis_name="c", subcore_axis_name="s")

    @functools.partial(
        pl.kernel,
        out_type=[
            jax.ShapeDtypeStruct((n_pad, d), jnp.float32),
            jax.ShapeDtypeStruct((n_pad, d), jnp.float32),
            jax.ShapeDtypeStruct((N_TILES, cpad), jnp.int32),
            jax.ShapeDtypeStruct((N_TILES, 8, LANES), jnp.float32),
        ],
        mesh=mesh,
        compiler_params=pltpu.CompilerParams(needs_layout_passes=False),
        scratch_types=[
            pltpu.VMEM((chunk,), jnp.int32),
            pltpu.VMEM((chunk,), jnp.int32),
            pltpu.VMEM((chunk,), jnp.int32),
            pltpu.VMEM((chunk,), jnp.int32),
            pltpu.VMEM((chunk,), jnp.int32),
            pltpu.VMEM((chunk + LANES,), jnp.int32),
            pltpu.VMEM((npt, d), jnp.float32),
            pltpu.VMEM((npt, d), jnp.float32),
            pltpu.VMEM((cpad,), jnp.int32),
            pltpu.VMEM((grp, 2 * d), jnp.float32),
            pltpu.VMEM((grp, 2 * d), jnp.float32),
            pltpu.VMEM((8, LANES), jnp.float32),
            pltpu.SemaphoreType.DMA,
            pltpu.SemaphoreType.DMA,
            pltpu.SemaphoreType.DMA,
            pltpu.SemaphoreType.DMA,
        ],
    )
    def sc_kernel(src_hbm, dst_hbm, t_hbm, m_out, s_out, cnt_out, part_out,
                  sbuf0, sbuf1, dbuf0, dbuf1, slist, dlist, accS, accM, cnt,
                  gbuf0, gbuf1, stage, sem_c0, sem_c1, sem_g0, sem_g1):
        wid = lax.axis_index("s") * 2 + lax.axis_index("c")
        base = wid * npt
        sbuf = (sbuf0, sbuf1)
        dbuf = (dbuf0, dbuf1)
        gbuf = (gbuf0, gbuf1)
        sem_c = (sem_c0, sem_c1)
        sem_g = (sem_g0, sem_g1)

        neg_inf = jnp.full((LANES,), -jnp.inf, dtype=jnp.float32)
        zerof = jnp.zeros((LANES,), jnp.float32)
        zeroi = jnp.zeros((LANES,), jnp.int32)
        one0 = jnp.where(lax.iota(jnp.int32, LANES) == 0, 1, 0)

        pltpu.async_copy(src_hbm.at[pl.ds(0, chunk)], sbuf[0], sem_c[0])
        pltpu.async_copy(dst_hbm.at[pl.ds(0, chunk)], dbuf[0], sem_c[0])

        def init_row(r, carry):
            for j in range(nb):
                accM[r, pl.ds(j * LANES, LANES)] = neg_inf
                accS[r, pl.ds(j * LANES, LANES)] = zerof
            return carry

        lax.fori_loop(0, npt, init_row, 0)

        def init_sl(i, carry):
            slist[pl.ds(i * LANES, LANES)] = zeroi
            return carry

        lax.fori_loop(0, chunk // LANES, init_sl, 0)

        def init_cnt(i, carry):
            cnt[pl.ds(i * LANES, LANES)] = zeroi
            return carry

        lax.fori_loop(0, cpad // LANES, init_cnt, 0)

        def chunk_pair(cp, carry):
            for b in range(2):
                ci = 2 * cp + b

                @pl.when(ci + 1 < n_chunks)
                def _():
                    nxt = pl.ds((ci + 1) * chunk, chunk)
                    pltpu.async_copy(src_hbm.at[nxt], sbuf[1 - b],
                                     sem_c[1 - b])
                    pltpu.async_copy(dst_hbm.at[nxt], dbuf[1 - b],
                                     sem_c[1 - b])

                cur = pl.ds(ci * chunk, chunk)
                pltpu.make_async_copy(src_hbm.at[cur], sbuf[b],
                                      sem_c[b]).wait()
                pltpu.make_async_copy(dst_hbm.at[cur], dbuf[b],
                                      sem_c[b]).wait()

                def filt(i, pos):
                    dv = dbuf[b][pl.ds(i * LANES, LANES)]
                    sv = sbuf[b][pl.ds(i * LANES, LANES)]
                    msk = (dv >= base) & (dv < base + npt)
                    pc = plsc.all_reduce_population_count(msk)

                    @pl.when(pc[0] > 0)
                    def _():
                        key = jnp.where(msk, 0, 1).astype(jnp.int32)
                        _, s_srt = plsc.sort_key_val(key, sv)
                        _, d_srt = plsc.sort_key_val(key, dv - base)
                        slist[pl.ds(pos, LANES)] = s_srt
                        dlist[pl.ds(pos, LANES)] = d_srt

                    return pos + pc[0]

                m_c = lax.fori_loop(0, chunk // LANES, filt, jnp.int32(0))
                ng = (m_c + (grp - 1)) // grp

                @pl.when(ng > 0)
                def _():
                    pltpu.async_copy(t_hbm.at[slist.at[pl.ds(0, grp)]],
                                     gbuf[0], sem_g[0])

                def group_pair(gp, carry2):
                    for gb in range(2):
                        g = 2 * gp + gb
                        g0 = g * grp

                        @pl.when(g + 1 < ng)
                        def _():
                            pltpu.async_copy(
                                t_hbm.at[slist.at[pl.ds(g0 + grp, grp)]],
                                gbuf[1 - gb], sem_g[1 - gb])

                        @pl.when(g < ng)
                        def _():
                            pltpu.make_async_copy(
                                t_hbm.at[slist.at[pl.ds(g0, grp)]],
                                gbuf[gb], sem_g[gb]).wait()

                        kn = jnp.clip(m_c - g0, 0, grp)

                        def edge_body(k, qs):
                            li = dlist[pl.ds(g0 + k, LANES)][0]
                            cnt[pl.ds(li, LANES)] = cnt[pl.ds(li, LANES)] + one0
                            for j in range(nb):
                                sl = pl.ds(j * LANES, LANES)
                                xp = gbuf[gb][k, sl]
                                accS[li, sl] = accS[li, sl] + xp
                                qs = qs + xp * xp
                                vp = gbuf[gb][k, pl.ds(d + j * LANES, LANES)]
                                accM[li, sl] = jnp.maximum(accM[li, sl], vp)
                            return qs

                        carry2 = lax.fori_loop(0, kn, edge_body, carry2)
                    return carry2

                carry = lax.fori_loop(0, (ng + 1) // 2, group_pair, carry)
            return carry

        qs = lax.fori_loop(0, n_chunks // 2, chunk_pair, zerof)

        stage[0, :] = qs
        for j in range(1, 8):
            stage[j, :] = zerof
        pltpu.sync_copy(stage, part_out.at[wid])
        pltpu.sync_copy(accM, m_out.at[pl.ds(base, npt)])
        pltpu.sync_copy(accS, s_out.at[pl.ds(base, npt)])
        pltpu.sync_copy(cnt, cnt_out.at[wid])

    return sc_kernel


def _stats_body(x_ref, s_ref, cnt_ref, part_ref, inv_ref, *, n_total):
    x = x_ref[...]
    s = s_ref[...]
    c = cnt_ref[...]
    qs = jnp.sum(part_ref[...][:, 0, :])
    s1 = jnp.sum(s) - jnp.sum(x * c)
    sq = qs + jnp.sum(x * x * c)
    cross = jnp.sum(x * s)
    s2 = sq - 2.0 * cross
    var = (s2 - s1 * s1 / n_total) / (n_total - 1.0)
    inv_ref[...] = jnp.reshape(1.0 / (jnp.sqrt(var) + 1e-5), (1, 1))


def _tc_stats(x, s, cnt, part, n_total):
    n, d = x.shape
    nt = part.shape[0]
    return pl.pallas_call(
        functools.partial(_stats_body, n_total=float(n_total)),
        in_specs=[
            pl.BlockSpec((n, d), lambda: (0, 0)),
            pl.BlockSpec((n, d), lambda: (0, 0)),
            pl.BlockSpec((n, 1), lambda: (0, 0)),
            pl.BlockSpec((nt, 8, LANES), lambda: (0, 0, 0)),
        ],
        out_specs=pl.BlockSpec((1, 1), lambda: (0, 0)),
        out_shape=jax.ShapeDtypeStruct((1, 1), jnp.float32),
    )(x, s, cnt, part)


def _fin_body(m_ref, u_ref, v_ref, inv_ref, cvec_ref, gam_ref, bet_ref,
              pa_ref, o_ref):
    inv = inv_ref[0, 0]
    m = m_ref[...]
    agg = u_ref[...] + cvec_ref[...] + inv * (m - v_ref[...])
    agg = jnp.where(m == -jnp.inf, 0.0, agg)
    mu = jnp.mean(agg, axis=-1, keepdims=True)
    dev = agg - mu
    va = jnp.mean(dev * dev, axis=-1, keepdims=True)
    h = dev * lax.rsqrt(va + 1e-5)
    h = h * gam_ref[...] + bet_ref[...]
    o_ref[...] = jnp.where(h >= 0.0, h, pa_ref[0, 0] * h)


def _tc_fin(m, u, v, inv, cvec, gam, bet, pa):
    n, d = u.shape
    rb = _row_block(n)
    return pl.pallas_call(
        _fin_body,
        grid=(n // rb,),
        in_specs=[
            pl.BlockSpec((rb, d), lambda i: (i, 0)),
            pl.BlockSpec((rb, d), lambda i: (i, 0)),
            pl.BlockSpec((rb, d), lambda i: (i, 0)),
            pl.BlockSpec((1, 1), lambda i: (0, 0)),
            pl.BlockSpec((1, d), lambda i: (0, 0)),
            pl.BlockSpec((1, d), lambda i: (0, 0)),
            pl.BlockSpec((1, d), lambda i: (0, 0)),
            pl.BlockSpec((1, 1), lambda i: (0, 0)),
        ],
        out_specs=pl.BlockSpec((rb, d), lambda i: (i, 0)),
        out_shape=jax.ShapeDtypeStruct((n, d), jnp.float32),
    )(m, u, v, inv, cvec, gam, bet, pa)


def kernel(x, edge_index, affine_w, affine_b, lin_W, lin_b, ln_gamma, ln_beta,
           prelu_a):
    n, d = x.shape
    e = edge_index.shape[1]
    src = edge_index[0].astype(jnp.int32)
    dst = edge_index[1].astype(jnp.int32)

    w1 = lin_W[:, :d]
    w2 = lin_W[:, d:]
    w1t = w1.T
    w2t = (w2 * affine_w[None, :]).T
    cvec = (affine_b @ w2.T + lin_b)[None, :]

    npt = (-(-n // N_TILES) + 7) // 8 * 8
    n_pad = N_TILES * npt
    chunk, grp = 2000, 32
    e_pad = -(-e // (2 * chunk)) * (2 * chunk)
    if e_pad != e:
        src = jnp.pad(src, (0, e_pad - e))
        dst = jnp.pad(dst, (0, e_pad - e), constant_values=jnp.int32(2 ** 30))

    t_tab, u = _tc_pre(x, w1t, w2t)
    m_full, s_full, cnt_full, part = _make_sc(e_pad, n_pad, d, npt, chunk,
                                              grp)(src, dst, t_tab)
    cnt = cnt_full[:, :npt].reshape(n_pad)[:n].astype(jnp.float32)[:, None]
    inv = _tc_stats(x, s_full[:n], cnt, part, e * d)
    out = _tc_fin(m_full[:n], u, t_tab[:, d:], inv, cvec,
                  ln_gamma[None, :], ln_beta[None, :],
                  jnp.reshape(prelu_a, (1, 1)), )
    return out


# speedup vs baseline: 1.3576x; 1.3576x over previous
import functools

import jax
import jax.numpy as jnp
from jax import lax
from jax.experimental import pallas as pl
from jax.experimental.pallas import tpu as pltpu
from jax.experimental.pallas import tpu_sc as plsc

N_TILES = 32
LANES = 16


def _row_block(n):
    for rb in (2000, 1000, 500, 250, 200, 125, 100, 50, 25, 10, 8, 5, 4, 2, 1):
        if n % rb == 0 and rb % 8 == 0 or n % rb == 0 and rb < 8:
            return rb
    return 1


def _pre_body(x_ref, w1t_ref, w2t_ref, t_ref, u_ref, xe_ref):
    xb = x_ref[...]
    rb, d = xb.shape
    t_ref[:, :d] = xb
    t_ref[:, d:] = jnp.dot(xb, w2t_ref[...], preferred_element_type=jnp.float32)
    u_ref[...] = jnp.dot(xb, w1t_ref[...], preferred_element_type=jnp.float32)
    rs = jnp.sum(xb, axis=1, keepdims=True)
    rq = jnp.sum(xb * xb, axis=1, keepdims=True)
    col = lax.broadcasted_iota(jnp.int32, (rb, LANES), 1)
    xe_ref[...] = jnp.where(col == 0, rs, jnp.where(col == 1, rq, 0.0))


def _tc_pre(x, w1t, w2t):
    n, d = x.shape
    rb = _row_block(n)
    return pl.pallas_call(
        _pre_body,
        grid=(n // rb,),
        in_specs=[
            pl.BlockSpec((rb, d), lambda i: (i, 0)),
            pl.BlockSpec((d, d), lambda i: (0, 0)),
            pl.BlockSpec((d, d), lambda i: (0, 0)),
        ],
        out_specs=[
            pl.BlockSpec((rb, 2 * d), lambda i: (i, 0)),
            pl.BlockSpec((rb, d), lambda i: (i, 0)),
            pl.BlockSpec((rb, LANES), lambda i: (i, 0)),
        ],
        out_shape=[
            jax.ShapeDtypeStruct((n, 2 * d), jnp.float32),
            jax.ShapeDtypeStruct((n, d), jnp.float32),
            jax.ShapeDtypeStruct((n, LANES), jnp.float32),
        ],
    )(x, w1t, w2t)


def _make_sc(e_pad, n_pad, d, npt, chunk, grp):
    nb = d // LANES
    n_chunks = e_pad // chunk
    mesh = plsc.VectorSubcoreMesh(core_axis_name="c", subcore_axis_name="s")

    @functools.partial(
        pl.kernel,
        out_type=[
            jax.ShapeDtypeStruct((n_pad, d), jnp.float32),
            jax.ShapeDtypeStruct((N_TILES, 8, LANES), jnp.float32),
        ],
        mesh=mesh,
        compiler_params=pltpu.CompilerParams(needs_layout_passes=False),
        scratch_types=[
            pltpu.VMEM((chunk,), jnp.int32),
            pltpu.VMEM((chunk,), jnp.int32),
            pltpu.VMEM((chunk,), jnp.int32),
            pltpu.VMEM((chunk,), jnp.int32),
            pltpu.VMEM((chunk,), jnp.int32),
            pltpu.VMEM((chunk + LANES,), jnp.int32),
            pltpu.VMEM((npt, d), jnp.float32),
            pltpu.VMEM((2 * npt + LANES,), jnp.float32),
            pltpu.VMEM((npt, d), jnp.float32),
            pltpu.VMEM((grp, 2 * d), jnp.float32),
            pltpu.VMEM((grp, 2 * d), jnp.float32),
            pltpu.VMEM((8, LANES), jnp.float32),
            pltpu.SemaphoreType.DMA,
            pltpu.SemaphoreType.DMA,
            pltpu.SemaphoreType.DMA,
            pltpu.SemaphoreType.DMA,
            pltpu.SemaphoreType.DMA,
        ],
    )
    def sc_kernel(src_hbm, dst_hbm, t_hbm, xpad_hbm, ext_hbm, m_out, part_out,
                  sbuf0, sbuf1, dbuf0, dbuf1, slist, dlist, xloc, xext, accM,
                  gbuf0, gbuf1, stage, sem_x, sem_c0, sem_c1, sem_g0, sem_g1):
        wid = lax.axis_index("s") * 2 + lax.axis_index("c")
        base = wid * npt
        sbuf = (sbuf0, sbuf1)
        dbuf = (dbuf0, dbuf1)
        gbuf = (gbuf0, gbuf1)
        sem_c = (sem_c0, sem_c1)
        sem_g = (sem_g0, sem_g1)

        neg_inf = jnp.full((LANES,), -jnp.inf, dtype=jnp.float32)
        zerof = jnp.zeros((LANES,), jnp.float32)
        zeroi = jnp.zeros((LANES,), jnp.int32)

        xcp = pltpu.async_copy(xpad_hbm.at[pl.ds(base, npt)], xloc, sem_x)
        ecp = pltpu.async_copy(
            ext_hbm.at[pl.ds(2 * base, 2 * npt + LANES)], xext, sem_x)
        pltpu.async_copy(src_hbm.at[pl.ds(0, chunk)], sbuf[0], sem_c[0])
        pltpu.async_copy(dst_hbm.at[pl.ds(0, chunk)], dbuf[0], sem_c[0])

        def init_row(r, carry):
            for j in range(nb):
                accM[r, pl.ds(j * LANES, LANES)] = neg_inf
            return carry

        lax.fori_loop(0, npt, init_row, 0)

        def init_sl(i, carry):
            slist[pl.ds(i * LANES, LANES)] = zeroi
            return carry

        lax.fori_loop(0, chunk // LANES, init_sl, 0)
        xcp.wait()
        ecp.wait()

        z8 = tuple(jnp.zeros((LANES,), jnp.float32) for _ in range(nb))

        def chunk_pair(cp, carry):
            for b in range(2):
                ci = 2 * cp + b

                @pl.when(ci + 1 < n_chunks)
                def _():
                    nxt = pl.ds((ci + 1) * chunk, chunk)
                    pltpu.async_copy(src_hbm.at[nxt], sbuf[1 - b],
                                     sem_c[1 - b])
                    pltpu.async_copy(dst_hbm.at[nxt], dbuf[1 - b],
                                     sem_c[1 - b])

                cur = pl.ds(ci * chunk, chunk)
                pltpu.make_async_copy(src_hbm.at[cur], sbuf[b],
                                      sem_c[b]).wait()
                pltpu.make_async_copy(dst_hbm.at[cur], dbuf[b],
                                      sem_c[b]).wait()

                def filt(i, pos):
                    dv = dbuf[b][pl.ds(i * LANES, LANES)]
                    sv = sbuf[b][pl.ds(i * LANES, LANES)]
                    msk = (dv >= base) & (dv < base + npt)
                    pc = plsc.all_reduce_population_count(msk)

                    @pl.when(pc[0] > 0)
                    def _():
                        key = jnp.where(msk, 0, 1).astype(jnp.int32)
                        _, s_srt = plsc.sort_key_val(key, sv)
                        _, d_srt = plsc.sort_key_val(key, dv - base)
                        slist[pl.ds(pos, LANES)] = s_srt
                        dlist[pl.ds(pos, LANES)] = d_srt

                    return pos + pc[0]

                m_c = lax.fori_loop(0, chunk // LANES, filt, jnp.int32(0))
                ng = (m_c + (grp - 1)) // grp

                @pl.when(ng > 0)
                def _():
                    pltpu.async_copy(t_hbm.at[slist.at[pl.ds(0, grp)]],
                                     gbuf[0], sem_g[0])

                def group_pair(gp, carry2):
                    for gb in range(2):
                        g = 2 * gp + gb
                        g0 = g * grp

                        @pl.when(g + 1 < ng)
                        def _():
                            pltpu.async_copy(
                                t_hbm.at[slist.at[pl.ds(g0 + grp, grp)]],
                                gbuf[1 - gb], sem_g[1 - gb])

                        @pl.when(g < ng)
                        def _():
                            pltpu.make_async_copy(
                                t_hbm.at[slist.at[pl.ds(g0, grp)]],
                                gbuf[gb], sem_g[gb]).wait()

                        kn = jnp.clip(m_c - g0, 0, grp)

                        def edge_body(k, carry3):
                            dacc, s2acc, crossacc, eacc = carry3
                            li = dlist[pl.ds(g0 + k, LANES)][0]
                            eacc = eacc + xext[pl.ds(2 * li, LANES)]
                            dn, qn, cn = [], [], []
                            for j in range(nb):
                                sl = pl.ds(j * LANES, LANES)
                                xp = gbuf[gb][k, sl]
                                xd = xloc[li, sl]
                                dn.append(dacc[j] + xp)
                                qn.append(s2acc[j] + xp * xp)
                                cn.append(crossacc[j] + xp * xd)
                                vp = gbuf[gb][k, pl.ds(d + j * LANES, LANES)]
                                accM[li, sl] = jnp.maximum(accM[li, sl], vp)
                            return (tuple(dn), tuple(qn), tuple(cn), eacc)

                        carry2 = lax.fori_loop(0, kn, edge_body, carry2)
                    return carry2

                carry = lax.fori_loop(0, (ng + 1) // 2, group_pair, carry)
            return carry

        dacc, s2acc, crossacc, eacc = lax.fori_loop(
            0, n_chunks // 2, chunk_pair, (z8, z8, z8, zerof))

        def vsum(acc):
            t = acc[0]
            for j in range(1, nb):
                t = t + acc[j]
            return t

        stage[0, :] = vsum(dacc)
        stage[1, :] = vsum(s2acc)
        stage[2, :] = vsum(crossacc)
        stage[3, :] = eacc
        for j in range(4, 8):
            stage[j, :] = zerof
        pltpu.sync_copy(stage, part_out.at[wid])
        pltpu.sync_copy(accM, m_out.at[pl.ds(base, npt)])

    return sc_kernel


def _fin_body(m_ref, u_ref, v_ref, part_ref, cvec_ref, gam_ref, bet_ref,
              pa_ref, o_ref, *, n_total):
    part = part_ref[...]
    s1 = jnp.sum(part[:, 0, :]) - jnp.sum(part[:, 3, 0])
    sq = jnp.sum(part[:, 1, :]) + jnp.sum(part[:, 3, 1])
    cross = jnp.sum(part[:, 2, :])
    s2 = sq - 2.0 * cross
    var = (s2 - s1 * s1 / n_total) / (n_total - 1.0)
    inv = 1.0 / (jnp.sqrt(var) + 1e-5)

    m = m_ref[...]
    agg = u_ref[...] + cvec_ref[...] + inv * (m - v_ref[...])
    agg = jnp.where(m == -jnp.inf, 0.0, agg)
    mu = jnp.mean(agg, axis=-1, keepdims=True)
    dev = agg - mu
    va = jnp.mean(dev * dev, axis=-1, keepdims=True)
    h = dev * lax.rsqrt(va + 1e-5)
    h = h * gam_ref[...] + bet_ref[...]
    o_ref[...] = jnp.where(h >= 0.0, h, pa_ref[0, 0] * h)


def _tc_fin(m, u, v, part, cvec, gam, bet, pa, n_total):
    n, d = u.shape
    rb = _row_block(n)
    nt = part.shape[0]
    return pl.pallas_call(
        functools.partial(_fin_body, n_total=float(n_total)),
        grid=(n // rb,),
        in_specs=[
            pl.BlockSpec((rb, d), lambda i: (i, 0)),
            pl.BlockSpec((rb, d), lambda i: (i, 0)),
            pl.BlockSpec((rb, d), lambda i: (i, 0)),
            pl.BlockSpec((nt, 8, LANES), lambda i: (0, 0, 0)),
            pl.BlockSpec((1, d), lambda i: (0, 0)),
            pl.BlockSpec((1, d), lambda i: (0, 0)),
            pl.BlockSpec((1, d), lambda i: (0, 0)),
            pl.BlockSpec((1, 1), lambda i: (0, 0)),
        ],
        out_specs=pl.BlockSpec((rb, d), lambda i: (i, 0)),
        out_shape=jax.ShapeDtypeStruct((n, d), jnp.float32),
    )(m, u, v, part, cvec, gam, bet, pa)


def kernel(x, edge_index, affine_w, affine_b, lin_W, lin_b, ln_gamma, ln_beta,
           prelu_a):
    n, d = x.shape
    e = edge_index.shape[1]
    src = edge_index[0].astype(jnp.int32)
    dst = edge_index[1].astype(jnp.int32)

    w1 = lin_W[:, :d]
    w2 = lin_W[:, d:]
    w1t = w1.T
    w2t = (w2 * affine_w[None, :]).T
    cvec = (affine_b @ w2.T + lin_b)[None, :]

    npt = (-(-n // N_TILES) + 7) // 8 * 8
    n_pad = N_TILES * npt
    chunk, grp = 2000, 24
    e_pad = -(-e // (2 * chunk)) * (2 * chunk)
    if e_pad != e:
        src = jnp.pad(src, (0, e_pad - e))
        dst = jnp.pad(dst, (0, e_pad - e), constant_values=jnp.int32(2 ** 30))
    t_tab, u, xe = _tc_pre(x, w1t, w2t)
    xpad = jnp.pad(x, ((0, n_pad - n), (0, 0)))
    ext_flat = jnp.pad(xe[:, :2].reshape(-1), (0, 2 * (n_pad - n) + LANES))
    m_full, part = _make_sc(e_pad, n_pad, d, npt, chunk, grp)(
        src, dst, t_tab, xpad, ext_flat)
    out = _tc_fin(m_full[:n], u, t_tab[:, d:], part, cvec,
                  ln_gamma[None, :], ln_beta[None, :],
                  jnp.reshape(prelu_a, (1, 1)), e * d)
    return out
